# baseline (speedup 1.0000x reference)
import functools

import jax
import jax.numpy as jnp
from jax import lax
from jax.experimental import pallas as pl
from jax.experimental.pallas import tpu as pltpu

N_NODES_C = 10000
N_REL_C = 3
N_EDGES_C = 320000
D_C = 128
NPAD = 10240
ROW_BLK = 1024


def _matmul_body(x_ref, w_ref, b_ref, y_ref):
    xb = x_ref[...]
    w = w_ref[0]
    acc = lax.dot_general(xb, w, (((1,), (1,)), ((), ())),
                          preferred_element_type=jnp.float32)
    y_ref[0] = acc + b_ref[0]


def _linear_all(x_pad, w_all, b_all):
    grid = (4, NPAD // ROW_BLK)
    return pl.pallas_call(
        _matmul_body,
        grid=grid,
        in_specs=[
            pl.BlockSpec((ROW_BLK, D_C), lambda j, i: (i, 0)),
            pl.BlockSpec((1, D_C, D_C), lambda j, i: (j, 0, 0)),
            pl.BlockSpec((1, 1, D_C), lambda j, i: (j, 0, 0)),
        ],
        out_specs=pl.BlockSpec((1, ROW_BLK, D_C), lambda j, i: (j, i, 0)),
        out_shape=jax.ShapeDtypeStruct((4, NPAD, D_C), jnp.float32),
    )(x_pad, w_all, b_all)


def kernel(x, edge_indices_list, W_root, b_root, W_rel):
    x_pad = jnp.pad(x, ((0, NPAD - N_NODES_C), (0, 0)))
    w_all = jnp.concatenate([W_root[None], W_rel], axis=0)
    b_all = jnp.concatenate([b_root[None], jnp.zeros((3, D_C), jnp.float32)],
                            0).reshape(4, 1, D_C)
    y = _linear_all(x_pad, w_all, b_all)
    out = y[0, :N_NODES_C]
    ei = edge_indices_list.astype(jnp.int32)
    for r in range(N_REL_C):
        h = y[r + 1, :N_NODES_C]
        src, dst = ei[r, 0], ei[r, 1]
        msg = jnp.take(h, src, axis=0)
        agg = jax.ops.segment_max(msg, dst, num_segments=N_NODES_C)
        agg = jnp.where(jnp.isneginf(agg), 0.0, agg)
        out = out + agg
    return out
